# baseline (speedup 1.0000x reference)
import jax
import jax.numpy as jnp
from jax import lax
from jax.experimental import pallas as pl
from jax.experimental.pallas import tpu as pltpu
from jax.experimental.pallas import tpu_sc as plsc

_E = 8
_D = 1024
_DFF = 1408
_T = 2048
_BLK = 128
_NTILES = _T * 2 // _BLK + _E
_CAP = _NTILES * _BLK
_NC = 2
_NS = 16
_NW = _NC * _NS
_TPW = _T // _NW
_CHW = 16


def _router_body(x_ref, gw_ref, pos1_ref, pos2_ref, w1_ref, w2_ref, te_ref):
    x = x_ref[...]
    gw = gw_ref[...]
    logits = lax.dot_general(x, gw, (((1,), (1,)), ((), ())),
                             preferred_element_type=jnp.float32)
    col = lax.broadcasted_iota(jnp.int32, (_T, _E), 1)
    m1 = jnp.max(logits, axis=1, keepdims=True)
    top1 = jnp.min(jnp.where(logits == m1, col, _E), axis=1, keepdims=True)
    oh1 = col == top1
    l2 = jnp.where(oh1, jnp.float32(-3.4e38), logits)
    m2 = jnp.max(l2, axis=1, keepdims=True)
    top2 = jnp.min(jnp.where(l2 == m2, col, _E), axis=1, keepdims=True)
    oh2 = col == top2
    w1 = jax.nn.sigmoid(m1 - m2)
    w1_ref[...] = jnp.broadcast_to(w1, (_T, 128))
    w2_ref[...] = jnp.broadcast_to(1.0 - w1, (_T, 128))

    inc = oh1.astype(jnp.float32) + oh2.astype(jnp.float32)
    c = jnp.concatenate([jnp.zeros((1, _E), jnp.float32), inc[:-1]], axis=0)
    k = 1
    while k < _T:
        c = c + jnp.concatenate(
            [jnp.zeros((k, _E), jnp.float32), c[:-k]], axis=0)
        k *= 2
    counts = jnp.sum(inc, axis=0, keepdims=True)
    gsize = jnp.ceil(counts / _BLK) * _BLK
    off = jnp.concatenate([jnp.zeros((1, 1), jnp.float32), gsize[:, :-1]],
                          axis=1)
    j = 1
    while j < _E:
        off = off + jnp.concatenate(
            [jnp.zeros((1, j), jnp.float32), off[:, :-j]], axis=1)
        j *= 2

    rank1 = jnp.sum(c * oh1, axis=1, keepdims=True)
    rank2 = jnp.sum(c * oh2, axis=1, keepdims=True)
    off1 = jnp.sum(off * oh1, axis=1, keepdims=True)
    off2 = jnp.sum(off * oh2, axis=1, keepdims=True)
    pos1_ref[...] = (off1 + rank1).astype(jnp.int32)
    pos2_ref[...] = (off2 + rank2).astype(jnp.int32)

    ends = off + gsize
    jrow = lax.broadcasted_iota(jnp.int32, (_NTILES, _E), 0) * _BLK
    te = jnp.sum((jrow >= ends.astype(jnp.int32)).astype(jnp.int32), axis=1,
                 keepdims=True)
    valid = (te < _E).astype(jnp.int32)
    te_ref[...] = jnp.concatenate([jnp.minimum(te, _E - 1), valid], axis=1)


def _router(x, gate_w):
    return pl.pallas_call(
        _router_body,
        out_shape=(
            jax.ShapeDtypeStruct((_T, 1), jnp.int32),
            jax.ShapeDtypeStruct((_T, 1), jnp.int32),
            jax.ShapeDtypeStruct((_T, 128), jnp.float32),
            jax.ShapeDtypeStruct((_T, 128), jnp.float32),
            jax.ShapeDtypeStruct((_NTILES, 2), jnp.int32),
        ),
    )(x, gate_w)


def _scatter_body(x_hbm, pos1_hbm, pos2_hbm, xs_hbm,
                  rows_v, idx1_v, idx2_v, sem):
    wid = lax.axis_index("s") * _NC + lax.axis_index("c")
    base = wid * _TPW
    pltpu.sync_copy(pos1_hbm.at[pl.ds(base, _TPW)], idx1_v)
    pltpu.sync_copy(pos2_hbm.at[pl.ds(base, _TPW)], idx2_v)
    pltpu.sync_copy(x_hbm.at[pl.ds(base, _TPW)], rows_v)
    c1 = pltpu.async_copy(rows_v, xs_hbm.at[idx1_v], sem)
    c2 = pltpu.async_copy(rows_v, xs_hbm.at[idx2_v], sem)
    c1.wait()
    c2.wait()


def _sc_scatter(x, pos1, pos2):
    mesh = plsc.VectorSubcoreMesh(core_axis_name="c", subcore_axis_name="s")
    return pl.kernel(
        _scatter_body,
        mesh=mesh,
        out_type=jax.ShapeDtypeStruct((_CAP, _D), jnp.float32),
        scratch_types=[
            pltpu.VMEM((_TPW, _D), jnp.float32),
            pltpu.VMEM((_TPW,), jnp.int32),
            pltpu.VMEM((_TPW,), jnp.int32),
            pltpu.SemaphoreType.DMA,
        ],
    )(x, pos1, pos2)


def _expert_body(te_ref, xs_ref, wgu_ref, wd_ref, out_ref, h_ref):
    i = pl.program_id(0)
    p = pl.program_id(1)
    valid = te_ref[i, 1] == 1

    @pl.when(valid & (p == 0))
    def _():
        gu = jnp.dot(xs_ref[...], wgu_ref[0],
                     preferred_element_type=jnp.float32)
        g = gu[:, :_DFF]
        u = gu[:, _DFF:]
        h_ref[...] = g * jax.nn.sigmoid(g) * u

    @pl.when(valid & (p == 1))
    def _():
        out_ref[...] = jnp.dot(h_ref[...], wd_ref[0],
                               preferred_element_type=jnp.float32)


def _grouped_mlp(te, xs, w_gate_up, w_down):
    grid_spec = pltpu.PrefetchScalarGridSpec(
        num_scalar_prefetch=1,
        grid=(_NTILES, 2),
        in_specs=[
            pl.BlockSpec((_BLK, _D), lambda i, p, te: (i, 0)),
            pl.BlockSpec((1, _D, 2 * _DFF),
                         lambda i, p, te: (te[i, 0], 0, 0)),
            pl.BlockSpec((1, _DFF, _D),
                         lambda i, p, te: (jnp.where(
                             p == 0,
                             te[jnp.maximum(i - 1, 0), 0],
                             te[i, 0]), 0, 0)),
        ],
        out_specs=pl.BlockSpec((_BLK, _D), lambda i, p, te: (i, 0)),
        scratch_shapes=[pltpu.VMEM((_BLK, _DFF), jnp.float32)],
    )
    return pl.pallas_call(
        _expert_body,
        grid_spec=grid_spec,
        out_shape=jax.ShapeDtypeStruct((_CAP, _D), jnp.float32),
        compiler_params=pltpu.CompilerParams(
            dimension_semantics=("arbitrary", "arbitrary")),
    )(te, xs, w_gate_up, w_down)


def _shared_body(x_ref, wgu_ref, wd_ref, out_ref):
    gu = jnp.dot(x_ref[...], wgu_ref[...], preferred_element_type=jnp.float32)
    g = gu[:, :_DFF]
    u = gu[:, _DFF:]
    h = g * jax.nn.sigmoid(g) * u
    out_ref[...] = jnp.dot(h, wd_ref[...], preferred_element_type=jnp.float32)


def _shared_mlp(x, ws_gate_up, ws_down):
    return pl.pallas_call(
        _shared_body,
        grid=(_T // _BLK,),
        in_specs=[
            pl.BlockSpec((_BLK, _D), lambda i: (i, 0)),
            pl.BlockSpec((_D, 2 * _DFF), lambda i: (0, 0)),
            pl.BlockSpec((_DFF, _D), lambda i: (0, 0)),
        ],
        out_specs=pl.BlockSpec((_BLK, _D), lambda i: (i, 0)),
        out_shape=jax.ShapeDtypeStruct((_T, _D), jnp.float32),
    )(x, ws_gate_up, ws_down)


def _combine_body(hbuf_hbm, sh_hbm, pos1_hbm, pos2_hbm, w1_hbm, w2_hbm,
                  out_hbm,
                  idx1_v, idx2_v, h1_v, h2_v, sh_v, w1_v, w2_v, sem):
    wid = lax.axis_index("s") * _NC + lax.axis_index("c")
    base = wid * _TPW
    nch = _TPW // _CHW

    def start(ci):
        b = ci % 2
        cb = base + ci * _CHW
        pltpu.sync_copy(pos1_hbm.at[pl.ds(cb, _CHW)], idx1_v.at[b])
        pltpu.sync_copy(pos2_hbm.at[pl.ds(cb, _CHW)], idx2_v.at[b])
        return (
            pltpu.async_copy(sh_hbm.at[pl.ds(cb, _CHW)], sh_v.at[b], sem),
            pltpu.async_copy(w1_hbm.at[pl.ds(cb, _CHW)], w1_v.at[b], sem),
            pltpu.async_copy(w2_hbm.at[pl.ds(cb, _CHW)], w2_v.at[b], sem),
            pltpu.async_copy(hbuf_hbm.at[idx1_v.at[b]], h1_v.at[b], sem),
            pltpu.async_copy(hbuf_hbm.at[idx2_v.at[b]], h2_v.at[b], sem),
        )

    cps = start(0)
    for ci in range(nch):
        b = ci % 2
        for cp in cps:
            cp.wait()
        if ci + 1 < nch:
            cps = start(ci + 1)

        def row(r, rcarry):
            wv1 = w1_v[b, r, pl.ds(0, 16)]
            wv2 = w2_v[b, r, pl.ds(0, 16)]
            for c in range(_D // 16):
                s = pl.ds(c * 16, 16)
                sh_v[b, r, s] = (sh_v[b, r, s]
                                 + wv1 * h1_v[b, r, s]
                                 + wv2 * h2_v[b, r, s])
            return rcarry

        lax.fori_loop(0, _CHW, row, 0)
        pltpu.sync_copy(sh_v.at[b], out_hbm.at[pl.ds(base + ci * _CHW, _CHW)])


def _sc_combine(hbuf, shared, pos1, pos2, w1b, w2b):
    mesh = plsc.VectorSubcoreMesh(core_axis_name="c", subcore_axis_name="s")
    return pl.kernel(
        _combine_body,
        mesh=mesh,
        out_type=jax.ShapeDtypeStruct((_T, _D), jnp.float32),
        scratch_types=[
            pltpu.VMEM((2, _CHW), jnp.int32),
            pltpu.VMEM((2, _CHW), jnp.int32),
            pltpu.VMEM((2, _CHW, _D), jnp.float32),
            pltpu.VMEM((2, _CHW, _D), jnp.float32),
            pltpu.VMEM((2, _CHW, _D), jnp.float32),
            pltpu.VMEM((2, _CHW, 128), jnp.float32),
            pltpu.VMEM((2, _CHW, 128), jnp.float32),
            pltpu.SemaphoreType.DMA,
        ],
    )(hbuf, shared, pos1, pos2, w1b, w2b)


def kernel(hidden_states, gate_w, w_gate_up, w_down, ws_gate_up, ws_down):
    x = hidden_states
    pos1, pos2, w1b, w2b, te = _router(x, gate_w)
    pos1 = pos1.reshape(_T)
    pos2 = pos2.reshape(_T)
    xs = _sc_scatter(x, pos1, pos2)
    shared = _shared_mlp(x, ws_gate_up, ws_down)
    hbuf = _grouped_mlp(te, xs, w_gate_up, w_down)
    return _sc_combine(hbuf, shared, pos1, pos2, w1b, w2b)


# speedup vs baseline: 1.0855x; 1.0855x over previous
import jax
import jax.numpy as jnp
from jax import lax
from jax.experimental import pallas as pl
from jax.experimental.pallas import tpu as pltpu
from jax.experimental.pallas import tpu_sc as plsc

_E = 8
_D = 1024
_DFF = 1408
_T = 2048
_BLK = 256
_NTILES = _T * 2 // _BLK + _E
_CAP = _NTILES * _BLK
_NC = 2
_NS = 16
_NW = _NC * _NS
_TPW = _T // _NW
_CHW = 16


def _router_body(x_ref, gw_ref, pos1_ref, pos2_ref, w1_ref, w2_ref, te_ref):
    x = x_ref[...]
    gw = gw_ref[...]
    logits = lax.dot_general(x, gw, (((1,), (1,)), ((), ())),
                             preferred_element_type=jnp.float32)
    col = lax.broadcasted_iota(jnp.int32, (_T, _E), 1)
    m1 = jnp.max(logits, axis=1, keepdims=True)
    top1 = jnp.min(jnp.where(logits == m1, col, _E), axis=1, keepdims=True)
    oh1 = col == top1
    l2 = jnp.where(oh1, jnp.float32(-3.4e38), logits)
    m2 = jnp.max(l2, axis=1, keepdims=True)
    top2 = jnp.min(jnp.where(l2 == m2, col, _E), axis=1, keepdims=True)
    oh2 = col == top2
    w1 = jax.nn.sigmoid(m1 - m2)
    w1_ref[...] = jnp.broadcast_to(w1, (_T, 128))
    w2_ref[...] = jnp.broadcast_to(1.0 - w1, (_T, 128))

    inc = oh1.astype(jnp.float32) + oh2.astype(jnp.float32)
    c = jnp.concatenate([jnp.zeros((1, _E), jnp.float32), inc[:-1]], axis=0)
    k = 1
    while k < _T:
        c = c + jnp.concatenate(
            [jnp.zeros((k, _E), jnp.float32), c[:-k]], axis=0)
        k *= 2
    counts = jnp.sum(inc, axis=0, keepdims=True)
    gsize = jnp.ceil(counts / _BLK) * _BLK
    off = jnp.concatenate([jnp.zeros((1, 1), jnp.float32), gsize[:, :-1]],
                          axis=1)
    j = 1
    while j < _E:
        off = off + jnp.concatenate(
            [jnp.zeros((1, j), jnp.float32), off[:, :-j]], axis=1)
        j *= 2

    rank1 = jnp.sum(c * oh1, axis=1, keepdims=True)
    rank2 = jnp.sum(c * oh2, axis=1, keepdims=True)
    off1 = jnp.sum(off * oh1, axis=1, keepdims=True)
    off2 = jnp.sum(off * oh2, axis=1, keepdims=True)
    pos1_ref[...] = (off1 + rank1).astype(jnp.int32)
    pos2_ref[...] = (off2 + rank2).astype(jnp.int32)

    ends = off + gsize
    jrow = lax.broadcasted_iota(jnp.int32, (_NTILES, _E), 0) * _BLK
    te = jnp.sum((jrow >= ends.astype(jnp.int32)).astype(jnp.int32), axis=1,
                 keepdims=True)
    valid = (te < _E).astype(jnp.int32)
    te_ref[...] = jnp.concatenate([jnp.minimum(te, _E - 1), valid], axis=1)


def _router(x, gate_w):
    return pl.pallas_call(
        _router_body,
        out_shape=(
            jax.ShapeDtypeStruct((_T, 1), jnp.int32),
            jax.ShapeDtypeStruct((_T, 1), jnp.int32),
            jax.ShapeDtypeStruct((_T, 128), jnp.float32),
            jax.ShapeDtypeStruct((_T, 128), jnp.float32),
            jax.ShapeDtypeStruct((_NTILES, 2), jnp.int32),
        ),
    )(x, gate_w)


def _scatter_body(x_hbm, pos1_hbm, pos2_hbm, xs_hbm,
                  rows_v, idx1_v, idx2_v, sem):
    wid = lax.axis_index("s") * _NC + lax.axis_index("c")
    base = wid * _TPW
    pltpu.sync_copy(pos1_hbm.at[pl.ds(base, _TPW)], idx1_v)
    pltpu.sync_copy(pos2_hbm.at[pl.ds(base, _TPW)], idx2_v)
    pltpu.sync_copy(x_hbm.at[pl.ds(base, _TPW)], rows_v)
    c1 = pltpu.async_copy(rows_v, xs_hbm.at[idx1_v], sem)
    c2 = pltpu.async_copy(rows_v, xs_hbm.at[idx2_v], sem)
    c1.wait()
    c2.wait()


def _sc_scatter(x, pos1, pos2):
    mesh = plsc.VectorSubcoreMesh(core_axis_name="c", subcore_axis_name="s")
    return pl.kernel(
        _scatter_body,
        mesh=mesh,
        out_type=jax.ShapeDtypeStruct((_CAP, _D), jnp.float32),
        scratch_types=[
            pltpu.VMEM((_TPW, _D), jnp.float32),
            pltpu.VMEM((_TPW,), jnp.int32),
            pltpu.VMEM((_TPW,), jnp.int32),
            pltpu.SemaphoreType.DMA,
        ],
    )(x, pos1, pos2)


def _expert_body(te_ref, xs_ref, wgu_ref, wd_ref, out_ref, h_ref):
    i = pl.program_id(0)
    p = pl.program_id(1)
    valid = te_ref[i, 1] == 1

    @pl.when(valid & (p == 0))
    def _():
        gu = jnp.dot(xs_ref[...], wgu_ref[0],
                     preferred_element_type=jnp.float32)
        g = gu[:, :_DFF]
        u = gu[:, _DFF:]
        h_ref[...] = g * jax.nn.sigmoid(g) * u

    @pl.when(valid & (p == 1))
    def _():
        out_ref[...] = jnp.dot(h_ref[...], wd_ref[0],
                               preferred_element_type=jnp.float32)


def _grouped_mlp(te, xs, w_gate_up, w_down):
    grid_spec = pltpu.PrefetchScalarGridSpec(
        num_scalar_prefetch=1,
        grid=(_NTILES, 2),
        in_specs=[
            pl.BlockSpec((_BLK, _D), lambda i, p, te: (i, 0)),
            pl.BlockSpec((1, _D, 2 * _DFF),
                         lambda i, p, te: (te[i, 0], 0, 0)),
            pl.BlockSpec((1, _DFF, _D),
                         lambda i, p, te: (jnp.where(
                             p == 0,
                             te[jnp.maximum(i - 1, 0), 0],
                             te[i, 0]), 0, 0)),
        ],
        out_specs=pl.BlockSpec((_BLK, _D), lambda i, p, te: (i, 0)),
        scratch_shapes=[pltpu.VMEM((_BLK, _DFF), jnp.float32)],
    )
    return pl.pallas_call(
        _expert_body,
        grid_spec=grid_spec,
        out_shape=jax.ShapeDtypeStruct((_CAP, _D), jnp.float32),
        compiler_params=pltpu.CompilerParams(
            dimension_semantics=("arbitrary", "arbitrary")),
    )(te, xs, w_gate_up, w_down)


def _shared_body(x_ref, wgu_ref, wd_ref, out_ref):
    gu = jnp.dot(x_ref[...], wgu_ref[...], preferred_element_type=jnp.float32)
    g = gu[:, :_DFF]
    u = gu[:, _DFF:]
    h = g * jax.nn.sigmoid(g) * u
    out_ref[...] = jnp.dot(h, wd_ref[...], preferred_element_type=jnp.float32)


def _shared_mlp(x, ws_gate_up, ws_down):
    return pl.pallas_call(
        _shared_body,
        grid=(_T // _BLK,),
        in_specs=[
            pl.BlockSpec((_BLK, _D), lambda i: (i, 0)),
            pl.BlockSpec((_D, 2 * _DFF), lambda i: (0, 0)),
            pl.BlockSpec((_DFF, _D), lambda i: (0, 0)),
        ],
        out_specs=pl.BlockSpec((_BLK, _D), lambda i: (i, 0)),
        out_shape=jax.ShapeDtypeStruct((_T, _D), jnp.float32),
    )(x, ws_gate_up, ws_down)


def _combine_body(hbuf_hbm, sh_hbm, pos1_hbm, pos2_hbm, w1_hbm, w2_hbm,
                  out_hbm,
                  idx1_v, idx2_v, h1_v, h2_v, sh_v, w1_v, w2_v, sem):
    wid = lax.axis_index("s") * _NC + lax.axis_index("c")
    base = wid * _TPW
    nch = _TPW // _CHW

    def start(ci):
        b = ci % 2
        cb = base + ci * _CHW
        pltpu.sync_copy(pos1_hbm.at[pl.ds(cb, _CHW)], idx1_v.at[b])
        pltpu.sync_copy(pos2_hbm.at[pl.ds(cb, _CHW)], idx2_v.at[b])
        return (
            pltpu.async_copy(sh_hbm.at[pl.ds(cb, _CHW)], sh_v.at[b], sem),
            pltpu.async_copy(w1_hbm.at[pl.ds(cb, _CHW)], w1_v.at[b], sem),
            pltpu.async_copy(w2_hbm.at[pl.ds(cb, _CHW)], w2_v.at[b], sem),
            pltpu.async_copy(hbuf_hbm.at[idx1_v.at[b]], h1_v.at[b], sem),
            pltpu.async_copy(hbuf_hbm.at[idx2_v.at[b]], h2_v.at[b], sem),
        )

    cps = start(0)
    for ci in range(nch):
        b = ci % 2
        for cp in cps:
            cp.wait()
        if ci + 1 < nch:
            cps = start(ci + 1)

        def row(r, rcarry):
            wv1 = w1_v[b, r, pl.ds(0, 16)]
            wv2 = w2_v[b, r, pl.ds(0, 16)]
            for c in range(_D // 16):
                s = pl.ds(c * 16, 16)
                sh_v[b, r, s] = (sh_v[b, r, s]
                                 + wv1 * h1_v[b, r, s]
                                 + wv2 * h2_v[b, r, s])
            return rcarry

        lax.fori_loop(0, _CHW, row, 0)
        pltpu.sync_copy(sh_v.at[b], out_hbm.at[pl.ds(base + ci * _CHW, _CHW)])


def _sc_combine(hbuf, shared, pos1, pos2, w1b, w2b):
    mesh = plsc.VectorSubcoreMesh(core_axis_name="c", subcore_axis_name="s")
    return pl.kernel(
        _combine_body,
        mesh=mesh,
        out_type=jax.ShapeDtypeStruct((_T, _D), jnp.float32),
        scratch_types=[
            pltpu.VMEM((2, _CHW), jnp.int32),
            pltpu.VMEM((2, _CHW), jnp.int32),
            pltpu.VMEM((2, _CHW, _D), jnp.float32),
            pltpu.VMEM((2, _CHW, _D), jnp.float32),
            pltpu.VMEM((2, _CHW, _D), jnp.float32),
            pltpu.VMEM((2, _CHW, 128), jnp.float32),
            pltpu.VMEM((2, _CHW, 128), jnp.float32),
            pltpu.SemaphoreType.DMA,
        ],
    )(hbuf, shared, pos1, pos2, w1b, w2b)


def kernel(hidden_states, gate_w, w_gate_up, w_down, ws_gate_up, ws_down):
    x = hidden_states
    pos1, pos2, w1b, w2b, te = _router(x, gate_w)
    pos1 = pos1.reshape(_T)
    pos2 = pos2.reshape(_T)
    xs = _sc_scatter(x, pos1, pos2)
    shared = _shared_mlp(x, ws_gate_up, ws_down)
    hbuf = _grouped_mlp(te, xs, w_gate_up, w_down)
    return _sc_combine(hbuf, shared, pos1, pos2, w1b, w2b)


# speedup vs baseline: 1.1372x; 1.0477x over previous
import jax
import jax.numpy as jnp
from jax import lax
from jax.experimental import pallas as pl
from jax.experimental.pallas import tpu as pltpu
from jax.experimental.pallas import tpu_sc as plsc

_E = 8
_D = 1024
_DFF = 1408
_T = 2048
_BLK = 256
_NTILES = _T * 2 // _BLK + _E
_CAP = _NTILES * _BLK
_NC = 2
_NS = 16
_NW = _NC * _NS
_TPW = _T // _NW
_CHW = 16


def _router_body(x_ref, gw_ref, pos1_ref, pos2_ref, w1_ref, w2_ref, te_ref):
    x = x_ref[...]
    gw = gw_ref[...]
    logits = lax.dot_general(x, gw, (((1,), (1,)), ((), ())),
                             preferred_element_type=jnp.float32)
    col = lax.broadcasted_iota(jnp.int32, (_T, _E), 1)
    m1 = jnp.max(logits, axis=1, keepdims=True)
    top1 = jnp.min(jnp.where(logits == m1, col, _E), axis=1, keepdims=True)
    oh1 = col == top1
    l2 = jnp.where(oh1, jnp.float32(-3.4e38), logits)
    m2 = jnp.max(l2, axis=1, keepdims=True)
    top2 = jnp.min(jnp.where(l2 == m2, col, _E), axis=1, keepdims=True)
    oh2 = col == top2
    w1 = jax.nn.sigmoid(m1 - m2)
    w1_ref[...] = jnp.broadcast_to(w1, (_T, 128))
    w2_ref[...] = jnp.broadcast_to(1.0 - w1, (_T, 128))

    inc = oh1.astype(jnp.float32) + oh2.astype(jnp.float32)
    c = jnp.concatenate([jnp.zeros((1, _E), jnp.float32), inc[:-1]], axis=0)
    k = 1
    while k < _T:
        c = c + jnp.concatenate(
            [jnp.zeros((k, _E), jnp.float32), c[:-k]], axis=0)
        k *= 2
    counts = jnp.sum(inc, axis=0, keepdims=True)
    gsize = jnp.ceil(counts / _BLK) * _BLK
    off = jnp.concatenate([jnp.zeros((1, 1), jnp.float32), gsize[:, :-1]],
                          axis=1)
    j = 1
    while j < _E:
        off = off + jnp.concatenate(
            [jnp.zeros((1, j), jnp.float32), off[:, :-j]], axis=1)
        j *= 2

    rank1 = jnp.sum(c * oh1, axis=1, keepdims=True)
    rank2 = jnp.sum(c * oh2, axis=1, keepdims=True)
    off1 = jnp.sum(off * oh1, axis=1, keepdims=True)
    off2 = jnp.sum(off * oh2, axis=1, keepdims=True)
    pos1_ref[...] = (off1 + rank1).astype(jnp.int32)
    pos2_ref[...] = (off2 + rank2).astype(jnp.int32)

    ends = off + gsize
    jrow = lax.broadcasted_iota(jnp.int32, (_NTILES, _E), 0) * _BLK
    te = jnp.sum((jrow >= ends.astype(jnp.int32)).astype(jnp.int32), axis=1,
                 keepdims=True)
    valid = (te < _E).astype(jnp.int32)
    te_ref[...] = jnp.concatenate([jnp.minimum(te, _E - 1), valid], axis=1)


def _router(x, gate_w):
    return pl.pallas_call(
        _router_body,
        out_shape=(
            jax.ShapeDtypeStruct((_T, 1), jnp.int32),
            jax.ShapeDtypeStruct((_T, 1), jnp.int32),
            jax.ShapeDtypeStruct((_T, 128), jnp.float32),
            jax.ShapeDtypeStruct((_T, 128), jnp.float32),
            jax.ShapeDtypeStruct((_NTILES, 2), jnp.int32),
        ),
    )(x, gate_w)


def _scatter_body(x_hbm, pos1_hbm, pos2_hbm, xs_hbm,
                  rows_v, idx1_v, idx2_v, sem):
    wid = lax.axis_index("s") * _NC + lax.axis_index("c")
    base = wid * _TPW
    pltpu.sync_copy(pos1_hbm.at[pl.ds(base, _TPW)], idx1_v)
    pltpu.sync_copy(pos2_hbm.at[pl.ds(base, _TPW)], idx2_v)
    pltpu.sync_copy(x_hbm.at[pl.ds(base, _TPW)], rows_v)
    c1 = pltpu.async_copy(rows_v, xs_hbm.at[idx1_v], sem)
    c2 = pltpu.async_copy(rows_v, xs_hbm.at[idx2_v], sem)
    c1.wait()
    c2.wait()


def _sc_scatter(x, pos1, pos2):
    mesh = plsc.VectorSubcoreMesh(core_axis_name="c", subcore_axis_name="s")
    return pl.kernel(
        _scatter_body,
        mesh=mesh,
        out_type=jax.ShapeDtypeStruct((_CAP, _D), jnp.float32),
        scratch_types=[
            pltpu.VMEM((_TPW, _D), jnp.float32),
            pltpu.VMEM((_TPW,), jnp.int32),
            pltpu.VMEM((_TPW,), jnp.int32),
            pltpu.SemaphoreType.DMA,
        ],
    )(x, pos1, pos2)


def _expert_body(te_ref, xs_ref, wgu_ref, wd_ref, out_ref):
    i = pl.program_id(0)

    @pl.when(te_ref[i, 1] == 1)
    def _():
        gu = jnp.dot(xs_ref[...], wgu_ref[0],
                     preferred_element_type=jnp.float32)
        g = gu[:, :_DFF]
        u = gu[:, _DFF:]
        h = g * jax.nn.sigmoid(g) * u
        out_ref[...] = jnp.dot(h, wd_ref[0], preferred_element_type=jnp.float32)


def _grouped_mlp(te, xs, w_gate_up, w_down):
    grid_spec = pltpu.PrefetchScalarGridSpec(
        num_scalar_prefetch=1,
        grid=(_NTILES,),
        in_specs=[
            pl.BlockSpec((_BLK, _D), lambda i, te: (i, 0)),
            pl.BlockSpec((1, _D, 2 * _DFF), lambda i, te: (te[i, 0], 0, 0)),
            pl.BlockSpec((1, _DFF, _D), lambda i, te: (te[i, 0], 0, 0)),
        ],
        out_specs=pl.BlockSpec((_BLK, _D), lambda i, te: (i, 0)),
    )
    return pl.pallas_call(
        _expert_body,
        grid_spec=grid_spec,
        out_shape=jax.ShapeDtypeStruct((_CAP, _D), jnp.float32),
        compiler_params=pltpu.CompilerParams(
            dimension_semantics=("arbitrary",)),
    )(te, xs, w_gate_up, w_down)


def _shared_body(x_ref, wgu_ref, wd_ref, out_ref):
    gu = jnp.dot(x_ref[...], wgu_ref[...], preferred_element_type=jnp.float32)
    g = gu[:, :_DFF]
    u = gu[:, _DFF:]
    h = g * jax.nn.sigmoid(g) * u
    out_ref[...] = jnp.dot(h, wd_ref[...], preferred_element_type=jnp.float32)


def _shared_mlp(x, ws_gate_up, ws_down):
    return pl.pallas_call(
        _shared_body,
        grid=(_T // _BLK,),
        in_specs=[
            pl.BlockSpec((_BLK, _D), lambda i: (i, 0)),
            pl.BlockSpec((_D, 2 * _DFF), lambda i: (0, 0)),
            pl.BlockSpec((_DFF, _D), lambda i: (0, 0)),
        ],
        out_specs=pl.BlockSpec((_BLK, _D), lambda i: (i, 0)),
        out_shape=jax.ShapeDtypeStruct((_T, _D), jnp.float32),
    )(x, ws_gate_up, ws_down)


def _combine_body(hbuf_hbm, sh_hbm, pos1_hbm, pos2_hbm, w1_hbm, w2_hbm,
                  out_hbm,
                  idx1_v, idx2_v, h1_v, h2_v, sh_v, w1_v, w2_v, sem):
    wid = lax.axis_index("s") * _NC + lax.axis_index("c")
    base = wid * _TPW
    nch = _TPW // _CHW

    def start(ci):
        b = ci % 2
        cb = base + ci * _CHW
        pltpu.sync_copy(pos1_hbm.at[pl.ds(cb, _CHW)], idx1_v.at[b])
        pltpu.sync_copy(pos2_hbm.at[pl.ds(cb, _CHW)], idx2_v.at[b])
        return (
            pltpu.async_copy(sh_hbm.at[pl.ds(cb, _CHW)], sh_v.at[b], sem),
            pltpu.async_copy(w1_hbm.at[pl.ds(cb, _CHW)], w1_v.at[b], sem),
            pltpu.async_copy(w2_hbm.at[pl.ds(cb, _CHW)], w2_v.at[b], sem),
            pltpu.async_copy(hbuf_hbm.at[idx1_v.at[b]], h1_v.at[b], sem),
            pltpu.async_copy(hbuf_hbm.at[idx2_v.at[b]], h2_v.at[b], sem),
        )

    cps = start(0)
    for ci in range(nch):
        b = ci % 2
        for cp in cps:
            cp.wait()
        if ci + 1 < nch:
            cps = start(ci + 1)

        def row(r, rcarry):
            wv1 = w1_v[b, r, pl.ds(0, 16)]
            wv2 = w2_v[b, r, pl.ds(0, 16)]
            for c in range(_D // 16):
                s = pl.ds(c * 16, 16)
                sh_v[b, r, s] = (sh_v[b, r, s]
                                 + wv1 * h1_v[b, r, s]
                                 + wv2 * h2_v[b, r, s])
            return rcarry

        lax.fori_loop(0, _CHW, row, 0)
        pltpu.sync_copy(sh_v.at[b], out_hbm.at[pl.ds(base + ci * _CHW, _CHW)])


def _sc_combine(hbuf, shared, pos1, pos2, w1b, w2b):
    mesh = plsc.VectorSubcoreMesh(core_axis_name="c", subcore_axis_name="s")
    return pl.kernel(
        _combine_body,
        mesh=mesh,
        out_type=jax.ShapeDtypeStruct((_T, _D), jnp.float32),
        scratch_types=[
            pltpu.VMEM((2, _CHW), jnp.int32),
            pltpu.VMEM((2, _CHW), jnp.int32),
            pltpu.VMEM((2, _CHW, _D), jnp.float32),
            pltpu.VMEM((2, _CHW, _D), jnp.float32),
            pltpu.VMEM((2, _CHW, _D), jnp.float32),
            pltpu.VMEM((2, _CHW, 128), jnp.float32),
            pltpu.VMEM((2, _CHW, 128), jnp.float32),
            pltpu.SemaphoreType.DMA,
        ],
    )(hbuf, shared, pos1, pos2, w1b, w2b)


def kernel(hidden_states, gate_w, w_gate_up, w_down, ws_gate_up, ws_down):
    x = hidden_states
    pos1, pos2, w1b, w2b, te = _router(x, gate_w)
    pos1 = pos1.reshape(_T)
    pos2 = pos2.reshape(_T)
    xs = _sc_scatter(x, pos1, pos2)
    shared = _shared_mlp(x, ws_gate_up, ws_down)
    hbuf = _grouped_mlp(te, xs, w_gate_up, w_down)
    return _sc_combine(hbuf, shared, pos1, pos2, w1b, w2b)


# speedup vs baseline: 1.2284x; 1.0801x over previous
import jax
import jax.numpy as jnp
from jax import lax
from jax.experimental import pallas as pl
from jax.experimental.pallas import tpu as pltpu
from jax.experimental.pallas import tpu_sc as plsc

_E = 8
_D = 1024
_DFF = 1408
_T = 2048
_BLK = 256
_NTILES = _T * 2 // _BLK + _E
_CAP = _NTILES * _BLK
_NC = 2
_NS = 16
_NW = _NC * _NS
_TPW = _T // _NW
_CHW = 16


def _router_body(x_ref, gw_ref, pos1_ref, pos2_ref, w1_ref, w2_ref, te_ref):
    x = x_ref[...]
    gw = gw_ref[...]
    logits = lax.dot_general(x, gw, (((1,), (1,)), ((), ())),
                             preferred_element_type=jnp.float32)
    col = lax.broadcasted_iota(jnp.int32, (_T, _E), 1)
    m1 = jnp.max(logits, axis=1, keepdims=True)
    top1 = jnp.min(jnp.where(logits == m1, col, _E), axis=1, keepdims=True)
    oh1 = col == top1
    l2 = jnp.where(oh1, jnp.float32(-3.4e38), logits)
    m2 = jnp.max(l2, axis=1, keepdims=True)
    top2 = jnp.min(jnp.where(l2 == m2, col, _E), axis=1, keepdims=True)
    oh2 = col == top2
    w1 = jax.nn.sigmoid(m1 - m2)
    w1_ref[...] = jnp.broadcast_to(w1, (_T, 128))
    w2_ref[...] = jnp.broadcast_to(1.0 - w1, (_T, 128))

    inc = oh1.astype(jnp.float32) + oh2.astype(jnp.float32)
    c = jnp.concatenate([jnp.zeros((1, _E), jnp.float32), inc[:-1]], axis=0)
    k = 1
    while k < _T:
        c = c + jnp.concatenate(
            [jnp.zeros((k, _E), jnp.float32), c[:-k]], axis=0)
        k *= 2
    counts = jnp.sum(inc, axis=0, keepdims=True)
    gsize = jnp.ceil(counts / _BLK) * _BLK
    off = jnp.concatenate([jnp.zeros((1, 1), jnp.float32), gsize[:, :-1]],
                          axis=1)
    j = 1
    while j < _E:
        off = off + jnp.concatenate(
            [jnp.zeros((1, j), jnp.float32), off[:, :-j]], axis=1)
        j *= 2

    rank1 = jnp.sum(c * oh1, axis=1, keepdims=True)
    rank2 = jnp.sum(c * oh2, axis=1, keepdims=True)
    off1 = jnp.sum(off * oh1, axis=1, keepdims=True)
    off2 = jnp.sum(off * oh2, axis=1, keepdims=True)
    pos1_ref[...] = (off1 + rank1).astype(jnp.int32)
    pos2_ref[...] = (off2 + rank2).astype(jnp.int32)

    ends = off + gsize
    jrow = lax.broadcasted_iota(jnp.int32, (_NTILES, _E), 0) * _BLK
    te = jnp.sum((jrow >= ends.astype(jnp.int32)).astype(jnp.int32), axis=1,
                 keepdims=True)
    valid = (te < _E).astype(jnp.int32)
    tec = jnp.minimum(te, _E - 1)
    prev = jnp.concatenate([tec[:1], tec[:-1]], axis=0)
    switch = (tec != prev).astype(jnp.int32)
    run = switch
    k = 1
    while k < _NTILES:
        run = run + jnp.concatenate(
            [jnp.zeros((k, 1), jnp.int32), run[:-k]], axis=0)
        k *= 2
    parity = (run + 1) % 2
    nrow = lax.broadcasted_iota(jnp.int32, (_NTILES, 1), 0)
    first = jnp.where(nrow == 0, 1, switch)
    col8b = lax.broadcasted_iota(jnp.int32, (_NTILES, _E), 1)
    gs_pos = gsize > 0
    nxt = jnp.min(jnp.where((col8b > tec) & gs_pos, col8b, _E), axis=1,
                  keepdims=True)
    fetch = first * (nxt < _E).astype(jnp.int32)
    zn = jnp.zeros((_NTILES, 1), jnp.int32)
    routed_sched = jnp.concatenate(
        [parity, first, fetch, jnp.minimum(nxt, _E - 1), valid, tec, zn, zn],
        axis=1)
    e0 = jnp.min(jnp.where(gs_pos, col8b[:1], _E), axis=1, keepdims=True)
    irow = lax.broadcasted_iota(jnp.int32, (8, 1), 0)
    one0 = (irow == 0).astype(jnp.int32)
    z8 = jnp.zeros((8, 1), jnp.int32)
    shared_sched = jnp.concatenate(
        [z8, one0, one0, jnp.broadcast_to(e0, (8, 1)), z8, z8, z8, z8],
        axis=1)
    te_ref[...] = jnp.concatenate([shared_sched, routed_sched], axis=0)


def _router(x, gate_w):
    return pl.pallas_call(
        _router_body,
        out_shape=(
            jax.ShapeDtypeStruct((_T, 1), jnp.int32),
            jax.ShapeDtypeStruct((_T, 1), jnp.int32),
            jax.ShapeDtypeStruct((_T, 128), jnp.float32),
            jax.ShapeDtypeStruct((_T, 128), jnp.float32),
            jax.ShapeDtypeStruct((_NTILES + 8, 8), jnp.int32),
        ),
    )(x, gate_w)


def _scatter_body(x_hbm, pos1_hbm, pos2_hbm, xs_hbm,
                  rows_v, idx1_v, idx2_v, sem):
    wid = lax.axis_index("s") * _NC + lax.axis_index("c")
    base = wid * _TPW
    pltpu.sync_copy(pos1_hbm.at[pl.ds(base, _TPW)], idx1_v)
    pltpu.sync_copy(pos2_hbm.at[pl.ds(base, _TPW)], idx2_v)
    pltpu.sync_copy(x_hbm.at[pl.ds(base, _TPW)], rows_v)
    c1 = pltpu.async_copy(rows_v, xs_hbm.at[idx1_v], sem)
    c2 = pltpu.async_copy(rows_v, xs_hbm.at[idx2_v], sem)
    c1.wait()
    c2.wait()


def _sc_scatter(x, pos1, pos2):
    mesh = plsc.VectorSubcoreMesh(core_axis_name="c", subcore_axis_name="s")
    return pl.kernel(
        _scatter_body,
        mesh=mesh,
        out_type=jax.ShapeDtypeStruct((_CAP, _D), jnp.float32),
        scratch_types=[
            pltpu.VMEM((_TPW, _D), jnp.float32),
            pltpu.VMEM((_TPW,), jnp.int32),
            pltpu.VMEM((_TPW,), jnp.int32),
            pltpu.SemaphoreType.DMA,
        ],
    )(x, pos1, pos2)


_NCHUNK = 4


def _wgu_copies(src, dst, sem):
    cs = _D // _NCHUNK
    return [pltpu.make_async_copy(src.at[pl.ds(k * cs, cs)],
                                  dst.at[pl.ds(k * cs, cs)], sem)
            for k in range(_NCHUNK)]


def _wd_copies(src, dst, sem):
    cs = _DFF // 2
    return [pltpu.make_async_copy(src.at[pl.ds(k * cs, cs)],
                                  dst.at[pl.ds(k * cs, cs)], sem)
            for k in range(2)]


def _mega_body(sched_ref, x_ref, xs_ref, wgu_any, wd_any, wsgu_any, wsd_any,
               out_ref, wgu_buf, wd_buf, sem):
    i = pl.program_id(0)
    par = sched_ref[i, 0]
    first = sched_ref[i, 1] == 1
    fetch = sched_ref[i, 2] == 1
    ne = sched_ref[i, 3]
    valid = sched_ref[i, 4] == 1

    @pl.when(i == 0)
    def _():
        for c in _wgu_copies(wsgu_any, wgu_buf.at[0], sem):
            c.start()
        for c in _wd_copies(wsd_any, wd_buf.at[0], sem):
            c.start()

    @pl.when(first)
    def _():
        for c in _wgu_copies(wsgu_any, wgu_buf.at[0], sem):
            c.wait()
        for c in _wd_copies(wsd_any, wd_buf.at[0], sem):
            c.wait()

    @pl.when(fetch)
    def _():
        pp = 1 - par
        for c in _wgu_copies(wgu_any.at[ne], wgu_buf.at[pp], sem):
            c.start()
        for c in _wd_copies(wd_any.at[ne], wd_buf.at[pp], sem):
            c.start()

    def mlp(in_ref, b):
        gu = jnp.dot(in_ref[...], wgu_buf[b],
                     preferred_element_type=jnp.float32)
        g = gu[:, :_DFF]
        u = gu[:, _DFF:]
        h = g * jax.nn.sigmoid(g) * u
        out_ref[...] = jnp.dot(h, wd_buf[b],
                               preferred_element_type=jnp.float32)

    @pl.when(i < 8)
    def _():
        mlp(x_ref, 0)

    @pl.when((i >= 8) & valid)
    def _():
        mlp(xs_ref, par)


def _mega_mlp(sched, x, xs, w_gate_up, w_down, ws_gate_up, ws_down):
    grid_spec = pltpu.PrefetchScalarGridSpec(
        num_scalar_prefetch=1,
        grid=(_NTILES + 8,),
        in_specs=[
            pl.BlockSpec((_BLK, _D), lambda i, s: (jnp.minimum(i, 7), 0)),
            pl.BlockSpec((_BLK, _D), lambda i, s: (jnp.maximum(i - 8, 0), 0)),
            pl.BlockSpec(memory_space=pl.ANY),
            pl.BlockSpec(memory_space=pl.ANY),
            pl.BlockSpec(memory_space=pl.ANY),
            pl.BlockSpec(memory_space=pl.ANY),
        ],
        out_specs=pl.BlockSpec(
            (_BLK, _D),
            lambda i, s: (jnp.where(i < 8, _CAP // _BLK + i, i - 8), 0)),
        scratch_shapes=[
            pltpu.VMEM((2, _D, 2 * _DFF), jnp.float32),
            pltpu.VMEM((2, _DFF, _D), jnp.float32),
            pltpu.SemaphoreType.DMA,
        ],
    )
    return pl.pallas_call(
        _mega_body,
        grid_spec=grid_spec,
        out_shape=jax.ShapeDtypeStruct((_CAP + _T, _D), jnp.float32),
        compiler_params=pltpu.CompilerParams(
            dimension_semantics=("arbitrary",)),
    )(sched, x, xs, w_gate_up, w_down, ws_gate_up, ws_down)


def _combine_body(hbuf_hbm, pos1_hbm, pos2_hbm, w1_hbm, w2_hbm,
                  out_hbm,
                  idx1_v, idx2_v, h1_v, h2_v, sh_v, w1_v, w2_v, sem):
    wid = lax.axis_index("s") * _NC + lax.axis_index("c")
    base = wid * _TPW
    nch = _TPW // _CHW

    def start(ci):
        b = ci % 2
        cb = base + ci * _CHW
        pltpu.sync_copy(pos1_hbm.at[pl.ds(cb, _CHW)], idx1_v.at[b])
        pltpu.sync_copy(pos2_hbm.at[pl.ds(cb, _CHW)], idx2_v.at[b])
        return (
            pltpu.async_copy(hbuf_hbm.at[pl.ds(_CAP + cb, _CHW)],
                             sh_v.at[b], sem),
            pltpu.async_copy(w1_hbm.at[pl.ds(cb, _CHW)], w1_v.at[b], sem),
            pltpu.async_copy(w2_hbm.at[pl.ds(cb, _CHW)], w2_v.at[b], sem),
            pltpu.async_copy(hbuf_hbm.at[idx1_v.at[b]], h1_v.at[b], sem),
            pltpu.async_copy(hbuf_hbm.at[idx2_v.at[b]], h2_v.at[b], sem),
        )

    cps = start(0)
    for ci in range(nch):
        b = ci % 2
        for cp in cps:
            cp.wait()
        if ci + 1 < nch:
            cps = start(ci + 1)

        def row(r, rcarry):
            wv1 = w1_v[b, r, pl.ds(0, 16)]
            wv2 = w2_v[b, r, pl.ds(0, 16)]
            for c in range(_D // 16):
                s = pl.ds(c * 16, 16)
                sh_v[b, r, s] = (sh_v[b, r, s]
                                 + wv1 * h1_v[b, r, s]
                                 + wv2 * h2_v[b, r, s])
            return rcarry

        lax.fori_loop(0, _CHW, row, 0)
        pltpu.sync_copy(sh_v.at[b], out_hbm.at[pl.ds(base + ci * _CHW, _CHW)])


def _sc_combine(hbuf, pos1, pos2, w1b, w2b):
    mesh = plsc.VectorSubcoreMesh(core_axis_name="c", subcore_axis_name="s")
    return pl.kernel(
        _combine_body,
        mesh=mesh,
        out_type=jax.ShapeDtypeStruct((_T, _D), jnp.float32),
        scratch_types=[
            pltpu.VMEM((2, _CHW), jnp.int32),
            pltpu.VMEM((2, _CHW), jnp.int32),
            pltpu.VMEM((2, _CHW, _D), jnp.float32),
            pltpu.VMEM((2, _CHW, _D), jnp.float32),
            pltpu.VMEM((2, _CHW, _D), jnp.float32),
            pltpu.VMEM((2, _CHW, 128), jnp.float32),
            pltpu.VMEM((2, _CHW, 128), jnp.float32),
            pltpu.SemaphoreType.DMA,
        ],
    )(hbuf, pos1, pos2, w1b, w2b)


def kernel(hidden_states, gate_w, w_gate_up, w_down, ws_gate_up, ws_down):
    x = hidden_states
    pos1, pos2, w1b, w2b, sched = _router(x, gate_w)
    pos1 = pos1.reshape(_T)
    pos2 = pos2.reshape(_T)
    xs = _sc_scatter(x, pos1, pos2)
    hbuf = _mega_mlp(sched, x, xs, w_gate_up, w_down, ws_gate_up, ws_down)
    return _sc_combine(hbuf, pos1, pos2, w1b, w2b)
